# baseline (speedup 1.0000x reference)
import functools

import jax
import jax.numpy as jnp
from jax import lax
from jax.experimental import pallas as pl
from jax.experimental.pallas import tpu as pltpu
from jax.experimental.pallas import tpu_sc as plsc

VOCAB = 1000000
EMB = 64
OUT = 64
B = 4096
L = 200

NC = 2
NS = 16
NW = NC * NS
E_PER_W = B // NW

CHUNKS = ((0, 120), (120, 80))


def _pool_body(x_hbm, table_hbm, out_hbm, idx_v, rows_v, pooled_v, sem):
    c = lax.axis_index("c")
    s = lax.axis_index("s")
    wid = s * NC + c
    base_e = wid * E_PER_W

    def elem(e, carry):
        flat = (base_e + e) * L
        pltpu.sync_copy(x_hbm.at[pl.ds(flat, L)], idx_v)
        cps = [
            pltpu.async_copy(
                table_hbm.at[idx_v.at[pl.ds(off, n)]],
                rows_v.at[pl.ds(off, n)],
                sem,
            )
            for off, n in CHUNKS
        ]
        for cp in cps:
            cp.wait()

        def red(r, accs):
            return tuple(
                accs[j] + rows_v[r, pl.ds(j * 16, 16)] for j in range(4)
            )

        z = jnp.zeros((16,), jnp.float32)
        acc = lax.fori_loop(0, L, red, (z, z, z, z))
        scale = jnp.float32(1.0 / L)
        for j in range(4):
            pooled_v[e, pl.ds(j * 16, 16)] = acc[j] * scale
        return carry

    lax.fori_loop(0, E_PER_W, elem, 0)
    pltpu.sync_copy(pooled_v, out_hbm.at[pl.ds(base_e, E_PER_W)])


@functools.partial(jax.jit, static_argnames=())
def _pool(x_flat, table):
    mesh = plsc.VectorSubcoreMesh(core_axis_name="c", subcore_axis_name="s")
    return pl.kernel(
        _pool_body,
        out_type=jax.ShapeDtypeStruct((B, EMB), jnp.float32),
        mesh=mesh,
        scratch_types=[
            pltpu.VMEM((L,), jnp.int32),
            pltpu.VMEM((L, EMB), jnp.float32),
            pltpu.VMEM((E_PER_W, EMB), jnp.float32),
            pltpu.SemaphoreType.DMA,
        ],
        compiler_params=pltpu.CompilerParams(use_tc_tiling_on_sc=False),
    )(x_flat, table)


def _mm_body(p_ref, w_ref, b_ref, o_ref):
    o_ref[...] = (
        jnp.dot(p_ref[...], w_ref[...], preferred_element_type=jnp.float32)
        + b_ref[...]
    )


def _matmul(pooled, W, b):
    return pl.pallas_call(
        _mm_body,
        out_shape=jax.ShapeDtypeStruct((B, OUT), jnp.float32),
    )(pooled, W, b.reshape(1, OUT))


def kernel(x, table, W, b):
    x_flat = x.reshape(-1).astype(jnp.int32)
    pooled = _pool(x_flat, table)
    return _matmul(pooled, W, b)


# speedup vs baseline: 1.2426x; 1.2426x over previous
import functools

import jax
import jax.numpy as jnp
from jax import lax
from jax.experimental import pallas as pl
from jax.experimental.pallas import tpu as pltpu
from jax.experimental.pallas import tpu_sc as plsc

VOCAB = 1000000
EMB = 64
OUT = 64
B = 4096
L = 200

NC = 2
NS = 16
NW = NC * NS
E_PER_W = B // NW

CHUNKS = ((0, 120), (120, 80))
UNROLL = 8


def _pool_body(x_hbm, table_hbm, out_hbm, idx_v, rows_v, pooled_v, sem0, sem1):
    c = lax.axis_index("c")
    s = lax.axis_index("s")
    wid = s * NC + c
    base_e = wid * E_PER_W
    sems = (sem0, sem1)
    last = jnp.int32(E_PER_W - 1)

    pltpu.sync_copy(x_hbm.at[pl.ds(base_e * L, E_PER_W * L)], idx_v)

    def fire(e, buf):
        for off, n in CHUNKS:
            pltpu.async_copy(
                table_hbm.at[idx_v.at[pl.ds(e * L + off, n)]],
                rows_v.at[buf].at[pl.ds(off, n)],
                sems[buf],
            )

    def wait(buf):
        for off, n in CHUNKS:
            pltpu.make_async_copy(
                table_hbm.at[idx_v.at[pl.ds(off, n)]],
                rows_v.at[buf].at[pl.ds(off, n)],
                sems[buf],
            ).wait()

    def reduce_into(e, buf):
        def red(i, accs):
            r = i * UNROLL
            out = list(accs)
            for rr in range(UNROLL):
                for j in range(4):
                    out[j] = out[j] + rows_v[buf, r + rr, pl.ds(j * 16, 16)]
            return tuple(out)

        z = jnp.zeros((16,), jnp.float32)
        acc = lax.fori_loop(0, L // UNROLL, red, (z,) * 4, unroll=1)
        scale = jnp.float32(1.0 / L)
        for j in range(4):
            pooled_v[e, pl.ds(j * 16, 16)] = acc[j] * scale

    fire(jnp.int32(0), 0)
    fire(jnp.int32(1), 1)

    def pair(i, carry):
        e0 = 2 * i
        wait(0)
        reduce_into(e0, 0)
        fire(jnp.minimum(e0 + 2, last), 0)
        wait(1)
        reduce_into(e0 + 1, 1)
        fire(jnp.minimum(e0 + 3, last), 1)
        return carry

    lax.fori_loop(0, E_PER_W // 2, pair, 0)
    wait(0)
    wait(1)
    pltpu.sync_copy(pooled_v, out_hbm.at[pl.ds(base_e, E_PER_W)])


@jax.jit
def _pool(x_flat, table):
    mesh = plsc.VectorSubcoreMesh(core_axis_name="c", subcore_axis_name="s")
    return pl.kernel(
        _pool_body,
        out_type=jax.ShapeDtypeStruct((B, EMB), jnp.float32),
        mesh=mesh,
        scratch_types=[
            pltpu.VMEM((E_PER_W * L,), jnp.int32),
            pltpu.VMEM((2, L, EMB), jnp.float32),
            pltpu.VMEM((E_PER_W, EMB), jnp.float32),
            pltpu.SemaphoreType.DMA,
            pltpu.SemaphoreType.DMA,
        ],
        compiler_params=pltpu.CompilerParams(use_tc_tiling_on_sc=False),
    )(x_flat, table)


def _mm_body(p_ref, w_ref, b_ref, o_ref):
    o_ref[...] = (
        jnp.dot(p_ref[...], w_ref[...], preferred_element_type=jnp.float32)
        + b_ref[...]
    )


def _matmul(pooled, W, b):
    return pl.pallas_call(
        _mm_body,
        out_shape=jax.ShapeDtypeStruct((B, OUT), jnp.float32),
    )(pooled, W, b.reshape(1, OUT))


def kernel(x, table, W, b):
    x_flat = x.reshape(-1).astype(jnp.int32)
    pooled = _pool(x_flat, table)
    return _matmul(pooled, W, b)
